# baseline (speedup 1.0000x reference)
import functools

import jax
import jax.numpy as jnp
from jax import lax
from jax.experimental import pallas as pl
from jax.experimental.pallas import tpu as pltpu
from jax.experimental.pallas import tpu_sc as plsc

B = 4096
DIM_CONTEXT = 256
EMB_DIM = 128
K = 200
KPAD = 208
KHALF = 104
LANES = 16
NC, NS = 2, 16
NW = NC * NS
ROWS_PER_W = B // NW
CHUNK = 64
NEG = -1e30


def _ctx_body(x_ref, w_ref, b_ref, o_ref):
    acc = jnp.dot(x_ref[...], w_ref[...], preferred_element_type=jnp.float32)
    o_ref[...] = jnp.maximum(acc + b_ref[...], 0.0).astype(jnp.bfloat16)


def _context(x, W, b):
    return pl.pallas_call(
        _ctx_body,
        grid=(B // 256,),
        in_specs=[
            pl.BlockSpec((256, DIM_CONTEXT), lambda i: (i, 0)),
            pl.BlockSpec((DIM_CONTEXT, EMB_DIM), lambda i: (0, 0)),
            pl.BlockSpec((1, EMB_DIM), lambda i: (0, 0)),
        ],
        out_specs=pl.BlockSpec((256, EMB_DIM), lambda i: (i, 0)),
        out_shape=jax.ShapeDtypeStruct((B, EMB_DIM), jnp.bfloat16),
    )(x, W, b.reshape(1, EMB_DIM))


def _sc_body(ctx_hbm, idx_hbm, table_hbm, out_hbm,
             idx_v, ctx_v, rows_v, out_v, scores_v, stage_v, sem0, sem1):
    wid = lax.axis_index("s") * NC + lax.axis_index("c")
    sems = (sem0, sem1)
    lane = lax.iota(jnp.int32, LANES)

    def issue(r, buf):
        sem = sems[buf]
        pltpu.async_copy(table_hbm.at[idx_v.at[r, pl.ds(0, KHALF)]],
                         rows_v.at[buf, pl.ds(0, KHALF)], sem)
        pltpu.async_copy(table_hbm.at[idx_v.at[r, pl.ds(KHALF, K - KHALF)]],
                         rows_v.at[buf, pl.ds(KHALF, K - KHALF)], sem)

    def drain(buf):
        pltpu.make_async_copy(table_hbm.at[pl.ds(0, K)],
                              rows_v.at[buf, pl.ds(0, K)], sems[buf]).wait()

    def compute(r, buf):
        ctx = [ctx_v[r, pl.ds(32 * c, 32)] for c in range(EMB_DIM // 32)]

        def kk_body(kk, mvec):
            kc = jnp.minimum(kk, KPAD // LANES - 1)
            slot = jnp.bitwise_and(kk, 1) * 256
            for j in range(LANES):
                k = kc * LANES + j
                prods = [rows_v[buf, k, pl.ds(32 * c, 32)] * ctx[c]
                         for c in range(EMB_DIM // 32)]
                while len(prods) > 1:
                    prods = [a + b for a, b in zip(prods[::2], prods[1::2])]
                ev, od = plsc.unpack(
                    prods[0], format=plsc.PackFormat.INTERLEAVED)
                stage_v[pl.ds(slot + LANES * j, LANES)] = plsc.cumsum(ev + od)
            hslot = jnp.bitwise_and(kk + 1, 1) * 256
            off = jnp.maximum(kk - 1, 0)
            svec = plsc.load_gather(
                stage_v, [hslot + lane * LANES + (LANES - 1)])
            svec = jnp.where((off * LANES + lane < K) & (kk > 0), svec, NEG)
            scores_v[pl.ds(off * LANES, LANES)] = svec
            return jnp.maximum(mvec, svec)

        mvec = lax.fori_loop(0, KPAD // LANES + 1, kk_body,
                             jnp.full((LANES,), NEG, jnp.float32))
        m = jnp.max(mvec)

        zacc = jnp.zeros((LANES,), jnp.float32)
        for c in range(KPAD // LANES):
            e = jnp.exp(scores_v[pl.ds(16 * c, 16)] - m)
            out_v[r, pl.ds(16 * c, 16)] = e
            zacc = zacc + e
        z = jnp.sum(zacc)
        inv = jnp.ones((LANES,), jnp.float32) / jnp.broadcast_to(z, (LANES,))
        for c in range(KPAD // LANES):
            out_v[r, pl.ds(16 * c, 16)] = out_v[r, pl.ds(16 * c, 16)] * inv

    for chunk in range(ROWS_PER_W // CHUNK):
        base = wid * ROWS_PER_W + chunk * CHUNK
        pltpu.sync_copy(idx_hbm.at[pl.ds(base, CHUNK)], idx_v)
        pltpu.sync_copy(ctx_hbm.at[pl.ds(base, CHUNK)], ctx_v)

        issue(0, 0)

        def g_body(g, carry):
            r0 = 2 * g
            issue(r0 + 1, 1)
            drain(0)
            compute(r0, 0)

            @pl.when(g < CHUNK // 2 - 1)
            def _():
                issue(r0 + 2, 0)

            drain(1)
            compute(r0 + 1, 1)
            return carry

        lax.fori_loop(0, CHUNK // 2, g_body, jnp.int32(0))
        pltpu.sync_copy(out_v.at[:, pl.ds(0, K)],
                        out_hbm.at[pl.ds(base, CHUNK)])


@functools.partial(
    pl.kernel,
    out_type=jax.ShapeDtypeStruct((B, K), jnp.float32),
    mesh=plsc.VectorSubcoreMesh(core_axis_name="c", subcore_axis_name="s",
                                num_cores=NC, num_subcores=NS),
    scratch_types=[
        pltpu.VMEM((CHUNK, K), jnp.int32),
        pltpu.VMEM((CHUNK, EMB_DIM), jnp.bfloat16),
        pltpu.VMEM((2, KPAD, EMB_DIM), jnp.bfloat16),
        pltpu.VMEM((CHUNK, KPAD), jnp.float32),
        pltpu.VMEM((KPAD,), jnp.float32),
        pltpu.VMEM((512,), jnp.float32),
        pltpu.SemaphoreType.DMA,
        pltpu.SemaphoreType.DMA,
    ],
    compiler_params=pltpu.CompilerParams(
        use_tc_tiling_on_sc=False, needs_layout_passes=False),
)
def _sc_softmax(ctx_hbm, idx_hbm, table_hbm, out_hbm, *scratch):
    _sc_body(ctx_hbm, idx_hbm, table_hbm, out_hbm, *scratch)


def kernel(x, A_k, W, b, table):
    ctx = _context(x, W, b)
    probs = _sc_softmax(ctx, A_k.astype(jnp.int32), table.astype(jnp.bfloat16))
    return probs


# speedup vs baseline: 2.2665x; 2.2665x over previous
import functools

import jax
import jax.numpy as jnp
from jax import lax
from jax.experimental import pallas as pl
from jax.experimental.pallas import tpu as pltpu
from jax.experimental.pallas import tpu_sc as plsc

B = 4096
DIM_CONTEXT = 256
EMB_DIM = 128
K = 200
KPAD = 208
KHALF = 104
LANES = 16
NC, NS = 2, 16
NW = NC * NS
ROWS_PER_W = B // NW
CHUNK = 64
NEG = -1e30


def _ctx_body(x_ref, w_ref, b_ref, o_ref):
    acc = jnp.dot(x_ref[...], w_ref[...], preferred_element_type=jnp.float32)
    o_ref[...] = jnp.maximum(acc + b_ref[...], 0.0).astype(jnp.bfloat16)


def _context(x, W, b):
    return pl.pallas_call(
        _ctx_body,
        grid=(B // 256,),
        in_specs=[
            pl.BlockSpec((256, DIM_CONTEXT), lambda i: (i, 0)),
            pl.BlockSpec((DIM_CONTEXT, EMB_DIM), lambda i: (0, 0)),
            pl.BlockSpec((1, EMB_DIM), lambda i: (0, 0)),
        ],
        out_specs=pl.BlockSpec((256, EMB_DIM), lambda i: (i, 0)),
        out_shape=jax.ShapeDtypeStruct((B, EMB_DIM), jnp.bfloat16),
    )(x, W, b.reshape(1, EMB_DIM))


def _sc_body(ctx_hbm, idx_hbm, table_hbm, out_hbm,
             idx_v, ctx_v, rows_v, out_v, scores_v, sem0, sem1):
    wid = lax.axis_index("s") * NC + lax.axis_index("c")
    sems = (sem0, sem1)
    lane = lax.iota(jnp.int32, LANES)

    def issue(r, buf):
        sem = sems[buf]
        pltpu.async_copy(table_hbm.at[idx_v.at[r, pl.ds(0, KHALF)]],
                         rows_v.at[buf, pl.ds(0, KHALF)], sem)
        pltpu.async_copy(table_hbm.at[idx_v.at[r, pl.ds(KHALF, K - KHALF)]],
                         rows_v.at[buf, pl.ds(KHALF, K - KHALF)], sem)

    def drain(buf):
        pltpu.make_async_copy(table_hbm.at[pl.ds(0, K)],
                              rows_v.at[buf, pl.ds(0, K)], sems[buf]).wait()

    def compute(r, buf):
        ctx = [ctx_v[r, pl.ds(32 * c, 32)] for c in range(EMB_DIM // 32)]

        def kk_body(kk, m):
            svec = jnp.zeros((LANES,), jnp.float32)
            for j in range(LANES):
                k = kk * LANES + j
                prods = [rows_v[buf, k, pl.ds(32 * c, 32)] * ctx[c]
                         for c in range(EMB_DIM // 32)]
                while len(prods) > 1:
                    prods = [a + b for a, b in zip(prods[::2], prods[1::2])]
                ev, od = plsc.unpack(
                    prods[0], format=plsc.PackFormat.INTERLEAVED)
                s = jnp.sum(ev + od)
                svec = jnp.where(lane == j, s, svec)
            svec = jnp.where(kk * LANES + lane < K, svec, NEG)
            scores_v[pl.ds(kk * LANES, LANES)] = svec
            return jnp.maximum(m, jnp.max(svec))

        m = lax.fori_loop(0, KPAD // LANES, kk_body, jnp.float32(NEG))

        zacc = jnp.zeros((LANES,), jnp.float32)
        for c in range(KPAD // LANES):
            e = jnp.exp(scores_v[pl.ds(16 * c, 16)] - m)
            out_v[r, pl.ds(16 * c, 16)] = e
            zacc = zacc + e
        z = jnp.sum(zacc)
        inv = jnp.ones((LANES,), jnp.float32) / jnp.broadcast_to(z, (LANES,))
        for c in range(KPAD // LANES):
            out_v[r, pl.ds(16 * c, 16)] = out_v[r, pl.ds(16 * c, 16)] * inv

    for chunk in range(ROWS_PER_W // CHUNK):
        base = wid * ROWS_PER_W + chunk * CHUNK
        pltpu.sync_copy(idx_hbm.at[pl.ds(base, CHUNK)], idx_v)
        pltpu.sync_copy(ctx_hbm.at[pl.ds(base, CHUNK)], ctx_v)

        issue(0, 0)

        def g_body(g, carry):
            r0 = 2 * g
            issue(r0 + 1, 1)
            drain(0)
            compute(r0, 0)

            @pl.when(g < CHUNK // 2 - 1)
            def _():
                issue(r0 + 2, 0)

            drain(1)
            compute(r0 + 1, 1)
            return carry

        lax.fori_loop(0, CHUNK // 2, g_body, jnp.int32(0))
        pltpu.sync_copy(out_v.at[:, pl.ds(0, K)],
                        out_hbm.at[pl.ds(base, CHUNK)])


@functools.partial(
    pl.kernel,
    out_type=jax.ShapeDtypeStruct((B, K), jnp.float32),
    mesh=plsc.VectorSubcoreMesh(core_axis_name="c", subcore_axis_name="s",
                                num_cores=NC, num_subcores=NS),
    scratch_types=[
        pltpu.VMEM((CHUNK, K), jnp.int32),
        pltpu.VMEM((CHUNK, EMB_DIM), jnp.bfloat16),
        pltpu.VMEM((2, KPAD, EMB_DIM), jnp.bfloat16),
        pltpu.VMEM((CHUNK, KPAD), jnp.float32),
        pltpu.VMEM((KPAD,), jnp.float32),
        pltpu.SemaphoreType.DMA,
        pltpu.SemaphoreType.DMA,
    ],
    compiler_params=pltpu.CompilerParams(
        use_tc_tiling_on_sc=False, needs_layout_passes=False),
)
def _sc_softmax(ctx_hbm, idx_hbm, table_hbm, out_hbm, *scratch):
    _sc_body(ctx_hbm, idx_hbm, table_hbm, out_hbm, *scratch)


def kernel(x, A_k, W, b, table):
    ctx = _context(x, W, b)
    probs = _sc_softmax(ctx, A_k.astype(jnp.int32), table.astype(jnp.bfloat16))
    return probs
